# baseline (speedup 1.0000x reference)
import functools

import jax
import jax.numpy as jnp
from jax import lax
from jax.experimental import pallas as pl
from jax.experimental.pallas import tpu as pltpu
from jax.experimental.pallas import tpu_sc as plsc

NC = 2
NS = 16
L = 16
NW = NC * NS

NVOX = 96 * 96 * 96
NPAIR = 16
TOTAL = NPAIR * NVOX
PER_W = TOTAL // NW
CH = 13824
NCHUNK = PER_W // CH
NCHUNK2 = NCHUNK // 2

NTHR = 32
NSEG = NTHR + 1
HSZ = 34 * L
NBANK = 8
HTOT = NBANK * HSZ


def _body(inp_hbm, tar_hbm, s_hbm, hist_out, sl1_out,
          s_v, x0, x1, t0, t1, hx, ht, sl1_v,
          sem_x0, sem_x1, sem_t0, sem_t1):
    wid = lax.axis_index("s") * NC + lax.axis_index("c")
    base = wid * PER_W

    pltpu.sync_copy(s_hbm, s_v)

    lane = lax.iota(jnp.int32, L)
    ones = jnp.full((L,), 1.0, jnp.float32)
    zero_f = jnp.zeros((L,), jnp.float32)

    def _zero(i, c):
        hx[pl.ds(i * L, L)] = zero_f
        ht[pl.ds(i * L, L)] = zero_f
        return c
    lax.fori_loop(0, HTOT // L, _zero, 0)

    s7 = plsc.load_gather(s_v, [jnp.full((L,), 7 * L, jnp.int32)])
    s15 = plsc.load_gather(s_v, [jnp.full((L,), 15 * L, jnp.int32)])
    s23 = plsc.load_gather(s_v, [jnp.full((L,), 23 * L, jnp.int32)])

    def rank16(v):
        m1 = s15 <= v
        pos = jnp.where(m1, jnp.int32(16 * L), jnp.int32(0))
        tv = jnp.where(m1, s23, s7)
        pos = pos + jnp.where(tv <= v, jnp.int32(8 * L), jnp.int32(0))
        for step in (4, 2, 1):
            tv = plsc.load_gather(s_v, [pos + jnp.int32((step - 1) * L)])
            pos = pos + jnp.where(tv <= v, jnp.int32(step * L), jnp.int32(0))
        tv = plsc.load_gather(s_v, [pos])
        return pos + jnp.where(tv <= v, jnp.int32(L), jnp.int32(0))

    def process(xr, tr, acc0):
        @plsc.parallel_loop(0, CH, L, unroll=NBANK, carry=acc0)
        def acc1(off, acc):
            bank = (lax.shift_right_logical(off, 4) & jnp.int32(NBANK - 1)) * jnp.int32(HSZ)
            x = xr[pl.ds(off, L)]
            t = tr[pl.ds(off, L)]
            d = x - t
            ad = jnp.abs(d)
            acc = acc + jnp.where(ad < 1.0, (0.5 * d) * d, ad - 0.5)
            blane = bank + lane
            plsc.addupdate_scatter(hx, [rank16(x) + blane], ones)
            plsc.addupdate_scatter(ht, [rank16(t) + blane], ones)
            return acc
        return acc1

    def start0(g2):
        pltpu.async_copy(inp_hbm.at[pl.ds(base + g2 * (2 * CH), CH)], x0, sem_x0)
        pltpu.async_copy(tar_hbm.at[pl.ds(base + g2 * (2 * CH), CH)], t0, sem_t0)

    def wait0():
        pltpu.make_async_copy(inp_hbm.at[pl.ds(base, CH)], x0, sem_x0).wait()
        pltpu.make_async_copy(tar_hbm.at[pl.ds(base, CH)], t0, sem_t0).wait()

    start0(0)

    def pair_body(g2, acc):
        off1 = base + g2 * (2 * CH) + CH
        wait0()
        h1 = pltpu.async_copy(inp_hbm.at[pl.ds(off1, CH)], x1, sem_x1)
        h2 = pltpu.async_copy(tar_hbm.at[pl.ds(off1, CH)], t1, sem_t1)
        acc = process(x0, t0, acc)

        @pl.when(g2 < NCHUNK2 - 1)
        def _():
            start0(g2 + 1)

        h1.wait()
        h2.wait()
        acc = process(x1, t1, acc)
        return acc

    acc = lax.fori_loop(0, NCHUNK2, pair_body, zero_f)

    sl1_v[...] = acc
    pltpu.sync_copy(sl1_v, sl1_out.at[wid])
    pltpu.sync_copy(hx, hist_out.at[wid, 0])
    pltpu.sync_copy(ht, hist_out.at[wid, 1])


@jax.jit
def kernel(inp, tar, bin_range):
    inp_f = inp.reshape(TOTAL)
    tar_f = tar.reshape(TOTAL)
    s = jnp.sort(bin_range.reshape(NTHR))
    s_rep = jnp.repeat(s, L)

    mesh = plsc.VectorSubcoreMesh(core_axis_name="c", subcore_axis_name="s")
    hist, sl1 = pl.kernel(
        _body,
        mesh=mesh,
        compiler_params=pltpu.CompilerParams(needs_layout_passes=False),
        out_type=[
            jax.ShapeDtypeStruct((NW, 2, HTOT), jnp.float32),
            jax.ShapeDtypeStruct((NW, L), jnp.float32),
        ],
        scratch_types=[
            pltpu.VMEM((NTHR * L,), jnp.float32),
            pltpu.VMEM((CH,), jnp.float32),
            pltpu.VMEM((CH,), jnp.float32),
            pltpu.VMEM((CH,), jnp.float32),
            pltpu.VMEM((CH,), jnp.float32),
            pltpu.VMEM((HTOT,), jnp.float32),
            pltpu.VMEM((HTOT,), jnp.float32),
            pltpu.VMEM((L,), jnp.float32),
            pltpu.SemaphoreType.DMA,
            pltpu.SemaphoreType.DMA,
            pltpu.SemaphoreType.DMA,
            pltpu.SemaphoreType.DMA,
        ],
    )(inp_f, tar_f, s_rep)

    h = hist.reshape(NW, 2, NBANK, HSZ)[:, :, :, : NSEG * L]
    h = h.reshape(NW, 2, NBANK, NSEG, L).sum(axis=(2, 4))
    h = h.reshape(NPAIR, 2, 2, NSEG).sum(axis=1)
    cdf = jnp.cumsum(h, axis=-1)
    plo = jnp.searchsorted(s, bin_range[:, 0], side="left")
    phi = jnp.searchsorted(s, bin_range[:, 1], side="left")
    cnt = jnp.maximum(cdf[:, :, phi] - cdf[:, :, plo], 0.0)
    freq = cnt / NVOX
    loss2 = jnp.mean(jnp.abs(freq[:, 0, :] - freq[:, 1, :]))
    loss1 = sl1.sum() / TOTAL
    return 0.5 * loss1 + 0.5 * loss2


# speedup vs baseline: 4.2659x; 4.2659x over previous
import functools

import jax
import jax.numpy as jnp
from jax import lax
from jax.experimental import pallas as pl
from jax.experimental.pallas import tpu as pltpu
from jax.experimental.pallas import tpu_sc as plsc

NC = 2
NS = 16
L = 16
NW = NC * NS

NVOX = 96 * 96 * 96
NPAIR = 16
TOTAL = NPAIR * NVOX
PER_W = TOTAL // NW
CH = 13824
NCHUNK = PER_W // CH
NCHUNK2 = NCHUNK // 2

NTHR = 32
NSEG = NTHR + 1
HSZ = 34 * L
NBANK = 8
HTOT = NBANK * HSZ


def _body(inp_hbm, tar_hbm, s_hbm, hist_out, sl1_out,
          s_v, x0, x1, t0, t1, hx, ht, sl1_v,
          sem_x0, sem_x1, sem_t0, sem_t1):
    wid = lax.axis_index("s") * NC + lax.axis_index("c")
    base = wid * PER_W

    pltpu.sync_copy(s_hbm, s_v)

    lane = lax.iota(jnp.int32, L)
    ones = jnp.full((L,), 1.0, jnp.float32)
    zero_f = jnp.zeros((L,), jnp.float32)

    def _zero(i, c):
        hx[pl.ds(i * L, L)] = zero_f
        ht[pl.ds(i * L, L)] = zero_f
        return c
    lax.fori_loop(0, HTOT // L, _zero, 0)

    sA = s_v[pl.ds(0, L)]
    sB = s_v[pl.ds(L, L)]
    idx15 = jnp.full((L,), 15, jnp.int32)
    s15v = jnp.take_along_axis(sA, idx15, axis=0, mode="promise_in_bounds")

    def rank(v):
        m1 = s15v <= v
        hi = jnp.where(m1, jnp.int32(16), jnp.int32(0))
        tbl = jnp.where(m1, sB, sA)
        tv = jnp.take_along_axis(tbl, jnp.full((L,), 7, jnp.int32), axis=0, mode="promise_in_bounds")
        pos = jnp.where(tv <= v, jnp.int32(8), jnp.int32(0))
        for step in (4, 2, 1):
            tv = jnp.take_along_axis(tbl, pos + jnp.int32(step - 1), axis=0, mode="promise_in_bounds")
            pos = pos + jnp.where(tv <= v, jnp.int32(step), jnp.int32(0))
        tv = jnp.take_along_axis(tbl, pos, axis=0, mode="promise_in_bounds")
        return hi + pos + jnp.where(tv <= v, jnp.int32(1), jnp.int32(0))

    def process(xr, tr, acc0):
        @plsc.parallel_loop(0, CH, L, unroll=NBANK, carry=acc0)
        def acc1(off, acc):
            bank = (lax.shift_right_logical(off, 4) & jnp.int32(NBANK - 1)) * jnp.int32(HSZ)
            x = xr[pl.ds(off, L)]
            t = tr[pl.ds(off, L)]
            d = x - t
            ad = jnp.abs(d)
            acc = acc + jnp.where(ad < 1.0, (0.5 * d) * d, ad - 0.5)
            blane = bank + lane
            plsc.addupdate_scatter(hx, [lax.shift_left(rank(x), 4) + blane], ones)
            plsc.addupdate_scatter(ht, [lax.shift_left(rank(t), 4) + blane], ones)
            return acc
        return acc1

    def start0(g2):
        pltpu.async_copy(inp_hbm.at[pl.ds(base + g2 * (2 * CH), CH)], x0, sem_x0)
        pltpu.async_copy(tar_hbm.at[pl.ds(base + g2 * (2 * CH), CH)], t0, sem_t0)

    def wait0():
        pltpu.make_async_copy(inp_hbm.at[pl.ds(base, CH)], x0, sem_x0).wait()
        pltpu.make_async_copy(tar_hbm.at[pl.ds(base, CH)], t0, sem_t0).wait()

    start0(0)

    def pair_body(g2, acc):
        off1 = base + g2 * (2 * CH) + CH
        wait0()
        h1 = pltpu.async_copy(inp_hbm.at[pl.ds(off1, CH)], x1, sem_x1)
        h2 = pltpu.async_copy(tar_hbm.at[pl.ds(off1, CH)], t1, sem_t1)
        acc = process(x0, t0, acc)

        @pl.when(g2 < NCHUNK2 - 1)
        def _():
            start0(g2 + 1)

        h1.wait()
        h2.wait()
        acc = process(x1, t1, acc)
        return acc

    acc = lax.fori_loop(0, NCHUNK2, pair_body, zero_f)

    sl1_v[...] = acc
    pltpu.sync_copy(sl1_v, sl1_out.at[wid])
    pltpu.sync_copy(hx, hist_out.at[wid, 0])
    pltpu.sync_copy(ht, hist_out.at[wid, 1])


@jax.jit
def kernel(inp, tar, bin_range):
    inp_f = inp.reshape(TOTAL)
    tar_f = tar.reshape(TOTAL)
    s = jnp.sort(bin_range.reshape(NTHR))

    mesh = plsc.VectorSubcoreMesh(core_axis_name="c", subcore_axis_name="s")
    hist, sl1 = pl.kernel(
        _body,
        mesh=mesh,
        compiler_params=pltpu.CompilerParams(needs_layout_passes=False),
        out_type=[
            jax.ShapeDtypeStruct((NW, 2, HTOT), jnp.float32),
            jax.ShapeDtypeStruct((NW, L), jnp.float32),
        ],
        scratch_types=[
            pltpu.VMEM((NTHR,), jnp.float32),
            pltpu.VMEM((CH,), jnp.float32),
            pltpu.VMEM((CH,), jnp.float32),
            pltpu.VMEM((CH,), jnp.float32),
            pltpu.VMEM((CH,), jnp.float32),
            pltpu.VMEM((HTOT,), jnp.float32),
            pltpu.VMEM((HTOT,), jnp.float32),
            pltpu.VMEM((L,), jnp.float32),
            pltpu.SemaphoreType.DMA,
            pltpu.SemaphoreType.DMA,
            pltpu.SemaphoreType.DMA,
            pltpu.SemaphoreType.DMA,
        ],
    )(inp_f, tar_f, s)

    h = hist.reshape(NW, 2, NBANK, HSZ)[:, :, :, : NSEG * L]
    h = h.reshape(NW, 2, NBANK, NSEG, L).sum(axis=(2, 4))
    h = h.reshape(NPAIR, 2, 2, NSEG).sum(axis=1)
    cdf = jnp.cumsum(h, axis=-1)
    plo = jnp.searchsorted(s, bin_range[:, 0], side="left")
    phi = jnp.searchsorted(s, bin_range[:, 1], side="left")
    cnt = jnp.maximum(cdf[:, :, phi] - cdf[:, :, plo], 0.0)
    freq = cnt / NVOX
    loss2 = jnp.mean(jnp.abs(freq[:, 0, :] - freq[:, 1, :]))
    loss1 = sl1.sum() / TOTAL
    return 0.5 * loss1 + 0.5 * loss2
